# Initial kernel scaffold; baseline (speedup 1.0000x reference)
import jax
import jax.numpy as jnp
from jax.experimental import pallas as pl


def kernel(x, edge_index, Wl1, bl1, Wr1, scale1, Wl2, bl2, Wr2, scale2):
    raise NotImplementedError("write your pallas kernel here")


# speedup vs baseline: 1.5509x; 1.5509x over previous
import functools

import jax
import jax.numpy as jnp
from jax import lax
from jax.experimental import pallas as pl
from jax.experimental.pallas import tpu as pltpu
from jax.experimental.pallas import tpu_sc as plsc

N = 10000
D = 128
E = 320000
EPS = 1e-12

NB = 10
BLK = N // NB

DSL = 4

CHUNK = 8000
NCHUNK = E // CHUNK
GROUPS = CHUNK // 16


def _inv_norm(x2):
    return lax.rsqrt(jnp.maximum(x2, EPS * EPS))


def _unshuffle(pe, po):
    blk = pe.shape[0]
    e3 = pe.reshape(blk, D // 8, 8)
    o3 = po.reshape(blk, D // 8, 8)
    return jnp.concatenate([e3[:, :, 4:8], o3[:, :, 0:4]], axis=-1).reshape(blk, D)


def _norm_body(x_ref, xn_ref):
    x = x_ref[...]
    n2 = jnp.sum(x * x, axis=1, keepdims=True)
    xn_ref[...] = x * _inv_norm(n2)


def _mid_body(x_ref, pe_ref, po_ref, wlts_ref, bl_ref, wrt_ref, h_ref, hn_ref):
    x = x_ref[...]
    n2 = jnp.sum(x * x, axis=1, keepdims=True)
    xn = x * _inv_norm(n2)
    agg = xn + _unshuffle(pe_ref[...], po_ref[...])
    a2 = jnp.sum(agg * agg, axis=1, keepdims=True)
    mn = agg * (_inv_norm(a2) * jnp.sqrt(n2))
    out = (jnp.dot(mn, wlts_ref[...], preferred_element_type=jnp.float32)
           + bl_ref[...]
           + jnp.dot(x, wrt_ref[...], preferred_element_type=jnp.float32))
    o2 = jnp.sum(out * out, axis=1, keepdims=True)
    h = jnp.maximum(out * _inv_norm(o2), 0.0)
    h_ref[...] = h
    h2 = jnp.sum(h * h, axis=1, keepdims=True)
    hn_ref[...] = h * _inv_norm(h2)


def _final_body(x_ref, pe_ref, po_ref, wlts_ref, bl_ref, wrt_ref, out_ref):
    x = x_ref[...]
    n2 = jnp.sum(x * x, axis=1, keepdims=True)
    xn = x * _inv_norm(n2)
    agg = xn + _unshuffle(pe_ref[...], po_ref[...])
    a2 = jnp.sum(agg * agg, axis=1, keepdims=True)
    mn = agg * (_inv_norm(a2) * jnp.sqrt(n2))
    out = (jnp.dot(mn, wlts_ref[...], preferred_element_type=jnp.float32)
           + bl_ref[...]
           + jnp.dot(x, wrt_ref[...], preferred_element_type=jnp.float32))
    o2 = jnp.sum(out * out, axis=1, keepdims=True)
    out_ref[...] = out * _inv_norm(o2)


_row_spec = pl.BlockSpec((BLK, D), lambda i: (i, 0))
_w_spec = pl.BlockSpec((D, D), lambda i: (0, 0))
_b_spec = pl.BlockSpec((1, D), lambda i: (0, 0))

_normalize = pl.pallas_call(
    _norm_body,
    grid=(NB,),
    in_specs=[_row_spec],
    out_specs=_row_spec,
    out_shape=jax.ShapeDtypeStruct((N, D), jnp.float32),
)

_mid = pl.pallas_call(
    _mid_body,
    grid=(NB,),
    in_specs=[_row_spec, _row_spec, _row_spec, _w_spec, _b_spec, _w_spec],
    out_specs=[_row_spec, _row_spec],
    out_shape=[jax.ShapeDtypeStruct((N, D), jnp.float32)] * 2,
)

_final = pl.pallas_call(
    _final_body,
    grid=(NB,),
    in_specs=[_row_spec, _row_spec, _row_spec, _w_spec, _b_spec, _w_spec],
    out_specs=_row_spec,
    out_shape=jax.ShapeDtypeStruct((N, D), jnp.float32),
)


def _scatter_body(xn_hbm, ei_hbm, oute_hbm, outo_hbm, win_t, src_b, dst_b):
    c = lax.axis_index("c")
    s = lax.axis_index("s")
    base = pl.multiple_of(s * 8, 8)
    goff = c * DSL
    aoff = DSL - goff

    pltpu.sync_copy(xn_hbm.at[:, pl.ds(base, 8)], win_t)

    lanes = lax.iota(jnp.int32, 16)
    r16 = lanes >> 2
    c16 = lanes & 3
    zeros = jnp.zeros((16,), jnp.float32)

    def zero_body(g, carry):
        plsc.store_scatter(win_t, [g * 4 + r16, c16 + aoff], zeros)
        return carry

    lax.fori_loop(0, N // 4, zero_body, 0)

    def chunk_body(ci, carry):
        coff = ci * CHUNK
        pltpu.sync_copy(ei_hbm.at[0, pl.ds(coff, CHUNK)], src_b)
        pltpu.sync_copy(ei_hbm.at[1, pl.ds(coff, CHUNK)], dst_b)

        def group_body(g, c2):
            b = g * 16
            src_v = src_b[pl.ds(b, 16)]
            dst_v = dst_b[pl.ds(b, 16)]
            for d in range(DSL):
                vals = plsc.load_gather(win_t, [src_v, jnp.full((16,), d, jnp.int32) + goff])
                plsc.addupdate_scatter(win_t, [dst_v, jnp.full((16,), d, jnp.int32) + aoff], vals)
            return c2

        lax.fori_loop(0, GROUPS, group_body, 0)
        return carry

    lax.fori_loop(0, NCHUNK, chunk_body, 0)

    @pl.when(c == 0)
    def _():
        pltpu.sync_copy(win_t, oute_hbm.at[:, pl.ds(base, 8)])

    @pl.when(c == 1)
    def _():
        pltpu.sync_copy(win_t, outo_hbm.at[:, pl.ds(base, 8)])


_sc_scatter = functools.partial(
    pl.kernel,
    out_type=[jax.ShapeDtypeStruct((N, D), jnp.float32)] * 2,
    mesh=plsc.VectorSubcoreMesh(core_axis_name="c", subcore_axis_name="s"),
    compiler_params=pltpu.CompilerParams(use_tc_tiling_on_sc=False,
                                         needs_layout_passes=False),
    scratch_types=[
        pltpu.VMEM((N, 8), jnp.float32),
        pltpu.VMEM((CHUNK,), jnp.int32),
        pltpu.VMEM((CHUNK,), jnp.int32),
    ],
)(_scatter_body)


def kernel(x, edge_index, Wl1, bl1, Wr1, scale1, Wl2, bl2, Wr2, scale2):
    wl1ts = (Wl1 * scale1).T
    wl2ts = (Wl2 * scale2).T
    wr1t = Wr1.T
    wr2t = Wr2.T
    bl1r = bl1.reshape(1, D)
    bl2r = bl2.reshape(1, D)

    xn = _normalize(x)
    pe1, po1 = _sc_scatter(xn, edge_index)
    h, hn = _mid(x, pe1, po1, wl1ts, bl1r, wr1t)
    pe2, po2 = _sc_scatter(hn, edge_index)
    return _final(h, pe2, po2, wl2ts, bl2r, wr2t)


# speedup vs baseline: 2.4145x; 1.5568x over previous
import functools

import jax
import jax.numpy as jnp
from jax import lax
from jax.experimental import pallas as pl
from jax.experimental.pallas import tpu as pltpu
from jax.experimental.pallas import tpu_sc as plsc

N = 10000
D = 128
E = 320000
EPS = 1e-12

NB = 10
BLK = N // NB

DSL = 4

CHUNK = 8000
NCHUNK = E // CHUNK
GROUPS = CHUNK // 16


def _inv_norm(x2):
    return lax.rsqrt(jnp.maximum(x2, EPS * EPS))


def _unshuffle(pe, po):
    blk = pe.shape[0]
    e3 = pe.reshape(blk, D // 8, 8)
    o3 = po.reshape(blk, D // 8, 8)
    return jnp.concatenate([e3[:, :, 4:8], o3[:, :, 0:4]], axis=-1).reshape(blk, D)


def _norm_body(x_ref, xn_ref):
    x = x_ref[...]
    n2 = jnp.sum(x * x, axis=1, keepdims=True)
    xn_ref[...] = x * _inv_norm(n2)


def _mid_body(x_ref, pe_ref, po_ref, wlts_ref, bl_ref, wrt_ref, h_ref, hn_ref):
    x = x_ref[...]
    n2 = jnp.sum(x * x, axis=1, keepdims=True)
    xn = x * _inv_norm(n2)
    agg = xn + _unshuffle(pe_ref[...], po_ref[...])
    a2 = jnp.sum(agg * agg, axis=1, keepdims=True)
    mn = agg * (_inv_norm(a2) * jnp.sqrt(n2))
    out = (jnp.dot(mn, wlts_ref[...], preferred_element_type=jnp.float32)
           + bl_ref[...]
           + jnp.dot(x, wrt_ref[...], preferred_element_type=jnp.float32))
    o2 = jnp.sum(out * out, axis=1, keepdims=True)
    h = jnp.maximum(out * _inv_norm(o2), 0.0)
    h_ref[...] = h
    h2 = jnp.sum(h * h, axis=1, keepdims=True)
    hn_ref[...] = h * _inv_norm(h2)


def _final_body(x_ref, pe_ref, po_ref, wlts_ref, bl_ref, wrt_ref, out_ref):
    x = x_ref[...]
    n2 = jnp.sum(x * x, axis=1, keepdims=True)
    xn = x * _inv_norm(n2)
    agg = xn + _unshuffle(pe_ref[...], po_ref[...])
    a2 = jnp.sum(agg * agg, axis=1, keepdims=True)
    mn = agg * (_inv_norm(a2) * jnp.sqrt(n2))
    out = (jnp.dot(mn, wlts_ref[...], preferred_element_type=jnp.float32)
           + bl_ref[...]
           + jnp.dot(x, wrt_ref[...], preferred_element_type=jnp.float32))
    o2 = jnp.sum(out * out, axis=1, keepdims=True)
    out_ref[...] = out * _inv_norm(o2)


_row_spec = pl.BlockSpec((BLK, D), lambda i: (i, 0))
_w_spec = pl.BlockSpec((D, D), lambda i: (0, 0))
_b_spec = pl.BlockSpec((1, D), lambda i: (0, 0))

_normalize = pl.pallas_call(
    _norm_body,
    grid=(NB,),
    in_specs=[_row_spec],
    out_specs=_row_spec,
    out_shape=jax.ShapeDtypeStruct((N, D), jnp.float32),
)

_mid = pl.pallas_call(
    _mid_body,
    grid=(NB,),
    in_specs=[_row_spec, _row_spec, _row_spec, _w_spec, _b_spec, _w_spec],
    out_specs=[_row_spec, _row_spec],
    out_shape=[jax.ShapeDtypeStruct((N, D), jnp.float32)] * 2,
)

_final = pl.pallas_call(
    _final_body,
    grid=(NB,),
    in_specs=[_row_spec, _row_spec, _row_spec, _w_spec, _b_spec, _w_spec],
    out_specs=_row_spec,
    out_shape=jax.ShapeDtypeStruct((N, D), jnp.float32),
)


def _scatter_body(xn_hbm, ei_hbm, oute_hbm, outo_hbm, win_t, src_b, dst_b):
    c = lax.axis_index("c")
    s = lax.axis_index("s")
    base = pl.multiple_of(s * 8, 8)
    goff = c * DSL
    aoff = DSL - goff

    pltpu.sync_copy(xn_hbm.at[:, pl.ds(base, 8)], win_t)

    lanes = lax.iota(jnp.int32, 16)
    r16 = lanes >> 2
    c16 = lanes & 3
    zeros = jnp.zeros((16,), jnp.float32)

    @plsc.parallel_loop(0, N // 4, unroll=8)
    def _(g):
        plsc.store_scatter(win_t, [g * 4 + r16, c16 + aoff], zeros)

    def chunk_body(ci, carry):
        coff = ci * CHUNK
        pltpu.sync_copy(ei_hbm.at[0, pl.ds(coff, CHUNK)], src_b)
        pltpu.sync_copy(ei_hbm.at[1, pl.ds(coff, CHUNK)], dst_b)

        @plsc.parallel_loop(0, CHUNK, step=16, unroll=8)
        def _(b):
            src_v = src_b[pl.ds(b, 16)]
            dst_v = dst_b[pl.ds(b, 16)]
            for d in range(DSL):
                vals = plsc.load_gather(win_t, [src_v, jnp.full((16,), d, jnp.int32) + goff])
                plsc.addupdate_scatter(win_t, [dst_v, jnp.full((16,), d, jnp.int32) + aoff], vals)

        return carry

    lax.fori_loop(0, NCHUNK, chunk_body, 0)

    @pl.when(c == 0)
    def _():
        pltpu.sync_copy(win_t, oute_hbm.at[:, pl.ds(base, 8)])

    @pl.when(c == 1)
    def _():
        pltpu.sync_copy(win_t, outo_hbm.at[:, pl.ds(base, 8)])


_sc_scatter = functools.partial(
    pl.kernel,
    out_type=[jax.ShapeDtypeStruct((N, D), jnp.float32)] * 2,
    mesh=plsc.VectorSubcoreMesh(core_axis_name="c", subcore_axis_name="s"),
    compiler_params=pltpu.CompilerParams(use_tc_tiling_on_sc=False,
                                         needs_layout_passes=False),
    scratch_types=[
        pltpu.VMEM((N, 8), jnp.float32),
        pltpu.VMEM((CHUNK,), jnp.int32),
        pltpu.VMEM((CHUNK,), jnp.int32),
    ],
)(_scatter_body)


def kernel(x, edge_index, Wl1, bl1, Wr1, scale1, Wl2, bl2, Wr2, scale2):
    wl1ts = (Wl1 * scale1).T
    wl2ts = (Wl2 * scale2).T
    wr1t = Wr1.T
    wr2t = Wr2.T
    bl1r = bl1.reshape(1, D)
    bl2r = bl2.reshape(1, D)

    xn = _normalize(x)
    pe1, po1 = _sc_scatter(xn, edge_index)
    h, hn = _mid(x, pe1, po1, wl1ts, bl1r, wr1t)
    pe2, po2 = _sc_scatter(hn, edge_index)
    return _final(h, pe2, po2, wl2ts, bl2r, wr2t)


# speedup vs baseline: 2.9938x; 1.2399x over previous
import functools

import jax
import jax.numpy as jnp
from jax import lax
from jax.experimental import pallas as pl
from jax.experimental.pallas import tpu as pltpu
from jax.experimental.pallas import tpu_sc as plsc

N = 10000
D = 128
E = 320000
EPS = 1e-12

NB = 10
BLK = N // NB

DSL = 4

CHUNK = 16000
NCHUNK = E // CHUNK
GROUPS = CHUNK // 16


def _inv_norm(x2):
    return lax.rsqrt(jnp.maximum(x2, EPS * EPS))


def _unshuffle(pe, po):
    blk = pe.shape[0]
    e3 = pe.reshape(blk, D // 8, 8)
    o3 = po.reshape(blk, D // 8, 8)
    return jnp.concatenate([e3[:, :, 4:8], o3[:, :, 0:4]], axis=-1).reshape(blk, D)


def _norm_body(x_ref, xn_ref):
    x = x_ref[...]
    n2 = jnp.sum(x * x, axis=1, keepdims=True)
    xn_ref[...] = x * _inv_norm(n2)


def _mid_body(x_ref, pe_ref, po_ref, wlts_ref, bl_ref, wrt_ref, h_ref, hn_ref):
    x = x_ref[...]
    n2 = jnp.sum(x * x, axis=1, keepdims=True)
    xn = x * _inv_norm(n2)
    agg = xn + _unshuffle(pe_ref[...], po_ref[...])
    a2 = jnp.sum(agg * agg, axis=1, keepdims=True)
    mn = agg * (_inv_norm(a2) * jnp.sqrt(n2))
    out = (jnp.dot(mn, wlts_ref[...], preferred_element_type=jnp.float32)
           + bl_ref[...]
           + jnp.dot(x, wrt_ref[...], preferred_element_type=jnp.float32))
    o2 = jnp.sum(out * out, axis=1, keepdims=True)
    h = jnp.maximum(out * _inv_norm(o2), 0.0)
    h_ref[...] = h
    h2 = jnp.sum(h * h, axis=1, keepdims=True)
    hn_ref[...] = h * _inv_norm(h2)


def _final_body(x_ref, pe_ref, po_ref, wlts_ref, bl_ref, wrt_ref, out_ref):
    x = x_ref[...]
    n2 = jnp.sum(x * x, axis=1, keepdims=True)
    xn = x * _inv_norm(n2)
    agg = xn + _unshuffle(pe_ref[...], po_ref[...])
    a2 = jnp.sum(agg * agg, axis=1, keepdims=True)
    mn = agg * (_inv_norm(a2) * jnp.sqrt(n2))
    out = (jnp.dot(mn, wlts_ref[...], preferred_element_type=jnp.float32)
           + bl_ref[...]
           + jnp.dot(x, wrt_ref[...], preferred_element_type=jnp.float32))
    o2 = jnp.sum(out * out, axis=1, keepdims=True)
    out_ref[...] = out * _inv_norm(o2)


def _pack_body(ei_ref, pk_ref):
    pk_ref[...] = ei_ref[0:1, :] | (ei_ref[1:2, :] << 16)


_pack_edges = pl.pallas_call(
    _pack_body,
    grid=(10,),
    in_specs=[pl.BlockSpec((2, E // 10), lambda i: (0, i))],
    out_specs=pl.BlockSpec((1, E // 10), lambda i: (0, i)),
    out_shape=jax.ShapeDtypeStruct((1, E), jnp.int32),
)


_row_spec = pl.BlockSpec((BLK, D), lambda i: (i, 0))
_w_spec = pl.BlockSpec((D, D), lambda i: (0, 0))
_b_spec = pl.BlockSpec((1, D), lambda i: (0, 0))

_normalize = pl.pallas_call(
    _norm_body,
    grid=(NB,),
    in_specs=[_row_spec],
    out_specs=_row_spec,
    out_shape=jax.ShapeDtypeStruct((N, D), jnp.float32),
)

_mid = pl.pallas_call(
    _mid_body,
    grid=(NB,),
    in_specs=[_row_spec, _row_spec, _row_spec, _w_spec, _b_spec, _w_spec],
    out_specs=[_row_spec, _row_spec],
    out_shape=[jax.ShapeDtypeStruct((N, D), jnp.float32)] * 2,
)

_final = pl.pallas_call(
    _final_body,
    grid=(NB,),
    in_specs=[_row_spec, _row_spec, _row_spec, _w_spec, _b_spec, _w_spec],
    out_specs=_row_spec,
    out_shape=jax.ShapeDtypeStruct((N, D), jnp.float32),
)


def _scatter_body(xn_hbm, pk_hbm, oute_hbm, outo_hbm, win_t, pk_b0, pk_b1, sem0, sem1):
    c = lax.axis_index("c")
    s = lax.axis_index("s")
    base = pl.multiple_of(s * 8, 8)
    goff = c * DSL
    aoff = DSL - goff

    first = pltpu.async_copy(pk_hbm.at[0, pl.ds(0, CHUNK)], pk_b0, sem0)
    pltpu.sync_copy(xn_hbm.at[:, pl.ds(base, 8)], win_t)

    lanes = lax.iota(jnp.int32, 16)
    r16 = lanes >> 2
    c16 = lanes & 3
    zeros = jnp.zeros((16,), jnp.float32)

    @plsc.parallel_loop(0, N // 4, unroll=8)
    def _(g):
        plsc.store_scatter(win_t, [g * 4 + r16, c16 + aoff], zeros)

    bufs = [pk_b0, pk_b1]
    sems = [sem0, sem1]
    copies = [first, None]
    for ci in range(NCHUNK):
        if ci + 1 < NCHUNK:
            copies[(ci + 1) % 2] = pltpu.async_copy(
                pk_hbm.at[0, pl.ds((ci + 1) * CHUNK, CHUNK)],
                bufs[(ci + 1) % 2], sems[(ci + 1) % 2])
        copies[ci % 2].wait()
        pk_b = bufs[ci % 2]

        @plsc.parallel_loop(0, CHUNK, step=16, unroll=16)
        def _(b):
            pk_v = pk_b[pl.ds(b, 16)]
            src_v = pk_v & 0xFFFF
            dst_v = pk_v >> 16
            for d in range(DSL):
                vals = plsc.load_gather(win_t, [src_v, jnp.full((16,), d, jnp.int32) + goff])
                plsc.addupdate_scatter(win_t, [dst_v, jnp.full((16,), d, jnp.int32) + aoff], vals)

    @pl.when(c == 0)
    def _():
        pltpu.sync_copy(win_t, oute_hbm.at[:, pl.ds(base, 8)])

    @pl.when(c == 1)
    def _():
        pltpu.sync_copy(win_t, outo_hbm.at[:, pl.ds(base, 8)])


_sc_scatter = functools.partial(
    pl.kernel,
    out_type=[jax.ShapeDtypeStruct((N, D), jnp.float32)] * 2,
    mesh=plsc.VectorSubcoreMesh(core_axis_name="c", subcore_axis_name="s"),
    compiler_params=pltpu.CompilerParams(use_tc_tiling_on_sc=False,
                                         needs_layout_passes=False),
    scratch_types=[
        pltpu.VMEM((N, 8), jnp.float32),
        pltpu.VMEM((CHUNK,), jnp.int32),
        pltpu.VMEM((CHUNK,), jnp.int32),
        pltpu.SemaphoreType.DMA,
        pltpu.SemaphoreType.DMA,
    ],
)(_scatter_body)


def kernel(x, edge_index, Wl1, bl1, Wr1, scale1, Wl2, bl2, Wr2, scale2):
    wl1ts = (Wl1 * scale1).T
    wl2ts = (Wl2 * scale2).T
    wr1t = Wr1.T
    wr2t = Wr2.T
    bl1r = bl1.reshape(1, D)
    bl2r = bl2.reshape(1, D)

    pk = _pack_edges(edge_index)
    xn = _normalize(x)
    pe1, po1 = _sc_scatter(xn, pk)
    h, hn = _mid(x, pe1, po1, wl1ts, bl1r, wr1t)
    pe2, po2 = _sc_scatter(hn, pk)
    return _final(h, pe2, po2, wl2ts, bl2r, wr2t)


# speedup vs baseline: 3.0581x; 1.0215x over previous
import functools

import jax
import jax.numpy as jnp
from jax import lax
from jax.experimental import pallas as pl
from jax.experimental.pallas import tpu as pltpu
from jax.experimental.pallas import tpu_sc as plsc

N = 10000
D = 128
E = 320000
EPS = 1e-12

NB = 10
BLK = N // NB

DSL = 4

CHUNK = 16000
NCHUNK = E // CHUNK
GROUPS = CHUNK // 16


def _inv_norm(x2):
    return lax.rsqrt(jnp.maximum(x2, EPS * EPS))


def _unshuffle(pe, po):
    blk = pe.shape[0]
    e3 = pe.reshape(blk, D // 8, 8)
    o3 = po.reshape(blk, D // 8, 8)
    return jnp.concatenate([e3[:, :, 4:8], o3[:, :, 0:4]], axis=-1).reshape(blk, D)


def _norm_body(x_ref, xn_ref):
    x = x_ref[...]
    n2 = jnp.sum(x * x, axis=1, keepdims=True)
    xn_ref[...] = x * _inv_norm(n2)


def _mid_body(x_ref, pe_ref, po_ref, wlts_ref, bl_ref, wrt_ref, h_ref, hn_ref):
    x = x_ref[...]
    n2 = jnp.sum(x * x, axis=1, keepdims=True)
    xn = x * _inv_norm(n2)
    agg = xn + _unshuffle(pe_ref[...], po_ref[...])
    a2 = jnp.sum(agg * agg, axis=1, keepdims=True)
    mn = agg * (_inv_norm(a2) * jnp.sqrt(n2))
    out = (jnp.dot(mn, wlts_ref[...], preferred_element_type=jnp.float32)
           + bl_ref[...]
           + jnp.dot(x, wrt_ref[...], preferred_element_type=jnp.float32))
    o2 = jnp.sum(out * out, axis=1, keepdims=True)
    h = jnp.maximum(out * _inv_norm(o2), 0.0)
    h_ref[...] = h
    h2 = jnp.sum(h * h, axis=1, keepdims=True)
    hn_ref[...] = h * _inv_norm(h2)


def _final_body(x_ref, pe_ref, po_ref, wlts_ref, bl_ref, wrt_ref, out_ref):
    x = x_ref[...]
    n2 = jnp.sum(x * x, axis=1, keepdims=True)
    xn = x * _inv_norm(n2)
    agg = xn + _unshuffle(pe_ref[...], po_ref[...])
    a2 = jnp.sum(agg * agg, axis=1, keepdims=True)
    mn = agg * (_inv_norm(a2) * jnp.sqrt(n2))
    out = (jnp.dot(mn, wlts_ref[...], preferred_element_type=jnp.float32)
           + bl_ref[...]
           + jnp.dot(x, wrt_ref[...], preferred_element_type=jnp.float32))
    o2 = jnp.sum(out * out, axis=1, keepdims=True)
    out_ref[...] = out * _inv_norm(o2)


def _pack_body(ei_ref, pk_ref):
    pk_ref[...] = ei_ref[0:1, :] | (ei_ref[1:2, :] << 16)


_pack_edges = pl.pallas_call(
    _pack_body,
    grid=(10,),
    in_specs=[pl.BlockSpec((2, E // 10), lambda i: (0, i))],
    out_specs=pl.BlockSpec((1, E // 10), lambda i: (0, i)),
    out_shape=jax.ShapeDtypeStruct((1, E), jnp.int32),
)


_row_spec = pl.BlockSpec((BLK, D), lambda i: (i, 0))
_w_spec = pl.BlockSpec((D, D), lambda i: (0, 0))
_b_spec = pl.BlockSpec((1, D), lambda i: (0, 0))

_normalize = pl.pallas_call(
    _norm_body,
    grid=(NB,),
    in_specs=[_row_spec],
    out_specs=_row_spec,
    out_shape=jax.ShapeDtypeStruct((N, D), jnp.float32),
)

_mid = pl.pallas_call(
    _mid_body,
    grid=(NB,),
    in_specs=[_row_spec, _row_spec, _row_spec, _w_spec, _b_spec, _w_spec],
    out_specs=[_row_spec, _row_spec],
    out_shape=[jax.ShapeDtypeStruct((N, D), jnp.float32)] * 2,
)

_final = pl.pallas_call(
    _final_body,
    grid=(NB,),
    in_specs=[_row_spec, _row_spec, _row_spec, _w_spec, _b_spec, _w_spec],
    out_specs=_row_spec,
    out_shape=jax.ShapeDtypeStruct((N, D), jnp.float32),
)


def _scatter_body(xn_hbm, pk_hbm, oute_hbm, outo_hbm, win_t, pk_b0, pk_b1, sem0, sem1):
    c = lax.axis_index("c")
    s = lax.axis_index("s")
    base = pl.multiple_of(s * 8, 8)
    goff = c * DSL
    aoff = DSL - goff

    first = pltpu.async_copy(pk_hbm.at[0, pl.ds(0, CHUNK)], pk_b0, sem0)
    pltpu.sync_copy(xn_hbm.at[:, pl.ds(base, 8)], win_t)

    lanes = lax.iota(jnp.int32, 16)
    r16 = lanes >> 2
    c16 = lanes & 3
    zeros = jnp.zeros((16,), jnp.float32)

    @plsc.parallel_loop(0, N // 4, unroll=8)
    def _(g):
        plsc.store_scatter(win_t, [g * 4 + r16, c16 + aoff], zeros)

    bufs = [pk_b0, pk_b1]
    sems = [sem0, sem1]
    copies = [first, None]
    for ci in range(NCHUNK):
        if ci + 1 < NCHUNK:
            copies[(ci + 1) % 2] = pltpu.async_copy(
                pk_hbm.at[0, pl.ds((ci + 1) * CHUNK, CHUNK)],
                bufs[(ci + 1) % 2], sems[(ci + 1) % 2])
        copies[ci % 2].wait()
        pk_b = bufs[ci % 2]

        gcols = [jnp.full((16,), d, jnp.int32) + goff for d in range(DSL)]
        acols = [jnp.full((16,), d, jnp.int32) + aoff for d in range(DSL)]

        @plsc.parallel_loop(0, CHUNK, step=16, unroll=32)
        def _(b):
            pk_v = pk_b[pl.ds(b, 16)]
            src_v = pk_v & 0xFFFF
            dst_v = pk_v >> 16
            for d in range(DSL):
                vals = plsc.load_gather(win_t, [src_v, gcols[d]])
                plsc.addupdate_scatter(win_t, [dst_v, acols[d]], vals)

    @pl.when(c == 0)
    def _():
        pltpu.sync_copy(win_t, oute_hbm.at[:, pl.ds(base, 8)])

    @pl.when(c == 1)
    def _():
        pltpu.sync_copy(win_t, outo_hbm.at[:, pl.ds(base, 8)])


_sc_scatter = functools.partial(
    pl.kernel,
    out_type=[jax.ShapeDtypeStruct((N, D), jnp.float32)] * 2,
    mesh=plsc.VectorSubcoreMesh(core_axis_name="c", subcore_axis_name="s"),
    compiler_params=pltpu.CompilerParams(use_tc_tiling_on_sc=False,
                                         needs_layout_passes=False),
    scratch_types=[
        pltpu.VMEM((N, 8), jnp.float32),
        pltpu.VMEM((CHUNK,), jnp.int32),
        pltpu.VMEM((CHUNK,), jnp.int32),
        pltpu.SemaphoreType.DMA,
        pltpu.SemaphoreType.DMA,
    ],
)(_scatter_body)


def kernel(x, edge_index, Wl1, bl1, Wr1, scale1, Wl2, bl2, Wr2, scale2):
    wl1ts = (Wl1 * scale1).T
    wl2ts = (Wl2 * scale2).T
    wr1t = Wr1.T
    wr2t = Wr2.T
    bl1r = bl1.reshape(1, D)
    bl2r = bl2.reshape(1, D)

    pk = _pack_edges(edge_index)
    xn = _normalize(x)
    pe1, po1 = _sc_scatter(xn, pk)
    h, hn = _mid(x, pe1, po1, wl1ts, bl1r, wr1t)
    pe2, po2 = _sc_scatter(hn, pk)
    return _final(h, pe2, po2, wl2ts, bl2r, wr2t)


# speedup vs baseline: 7.4882x; 2.4486x over previous
import functools

import jax
import jax.numpy as jnp
from jax import lax
from jax.experimental import pallas as pl
from jax.experimental.pallas import tpu as pltpu
from jax.experimental.pallas import tpu_sc as plsc

N = 10000
D = 128
E = 320000
EPS = 1e-12

NWORKERS = 32
DSL = D // NWORKERS

CHUNK = 16000
NCHUNK = E // CHUNK


def _inv_norm(x2):
    return lax.rsqrt(jnp.maximum(x2, EPS * EPS))


def _eye():
    r = lax.broadcasted_iota(jnp.int32, (D, D), 0)
    c = lax.broadcasted_iota(jnp.int32, (D, D), 1)
    return (r == c).astype(jnp.float32)


def _t_out(m):
    return lax.dot_general(_eye(), m, (((1,), (1,)), ((), ())),
                           preferred_element_type=jnp.float32)


def _t_in(mt):
    return lax.dot_general(mt, _eye(), (((0,), (0,)), ((), ())),
                           preferred_element_type=jnp.float32)


def _pack_body(ei_ref, pk_ref):
    pk_ref[...] = ei_ref[0:1, :] | (ei_ref[1:2, :] << 16)


_pack_edges = pl.pallas_call(
    _pack_body,
    grid=(10,),
    in_specs=[pl.BlockSpec((2, E // 10), lambda i: (0, i))],
    out_specs=pl.BlockSpec((1, E // 10), lambda i: (0, i)),
    out_shape=jax.ShapeDtypeStruct((1, E), jnp.int32),
)


def _norm_body(x_ref, xnt_ref):
    x = x_ref[...]
    n2 = jnp.sum(x * x, axis=1, keepdims=True)
    xnt_ref[...] = _t_out(x * _inv_norm(n2))


def _mid_body(x_ref, pt_ref, wlts_ref, bl_ref, wrt_ref, h_ref, hnt_ref):
    x = x_ref[...]
    n2 = jnp.sum(x * x, axis=1, keepdims=True)
    xn = x * _inv_norm(n2)
    agg = xn + _t_in(pt_ref[...])
    a2 = jnp.sum(agg * agg, axis=1, keepdims=True)
    mn = agg * (_inv_norm(a2) * jnp.sqrt(n2))
    out = (jnp.dot(mn, wlts_ref[...], preferred_element_type=jnp.float32)
           + bl_ref[...]
           + jnp.dot(x, wrt_ref[...], preferred_element_type=jnp.float32))
    o2 = jnp.sum(out * out, axis=1, keepdims=True)
    h = jnp.maximum(out * _inv_norm(o2), 0.0)
    h_ref[...] = h
    h2 = jnp.sum(h * h, axis=1, keepdims=True)
    hnt_ref[...] = _t_out(h * _inv_norm(h2))


def _final_body(x_ref, pt_ref, wlts_ref, bl_ref, wrt_ref, out_ref):
    x = x_ref[...]
    n2 = jnp.sum(x * x, axis=1, keepdims=True)
    xn = x * _inv_norm(n2)
    agg = xn + _t_in(pt_ref[...])
    a2 = jnp.sum(agg * agg, axis=1, keepdims=True)
    mn = agg * (_inv_norm(a2) * jnp.sqrt(n2))
    out = (jnp.dot(mn, wlts_ref[...], preferred_element_type=jnp.float32)
           + bl_ref[...]
           + jnp.dot(x, wrt_ref[...], preferred_element_type=jnp.float32))
    o2 = jnp.sum(out * out, axis=1, keepdims=True)
    out_ref[...] = out * _inv_norm(o2)


_nat = jax.ShapeDtypeStruct((N, D), jnp.float32)
_tr = jax.ShapeDtypeStruct((D, N), jnp.float32)

_normalize = pl.pallas_call(_norm_body, out_shape=_tr)

_mid = pl.pallas_call(_mid_body, out_shape=[_nat, _tr])

_final = pl.pallas_call(_final_body, out_shape=_nat)


def _scatter_body(xnt_hbm, pk_hbm, out_hbm, xn_t, acc_t, pk_b0, pk_b1, sem0, sem1):
    c = lax.axis_index("c")
    s = lax.axis_index("s")
    wid = s * 2 + c
    d0 = wid * DSL

    first = pltpu.async_copy(pk_hbm.at[0, pl.ds(0, CHUNK)], pk_b0, sem0)
    pltpu.sync_copy(xnt_hbm.at[pl.ds(d0, DSL), :], xn_t)

    zeros = jnp.zeros((16,), jnp.float32)

    @plsc.parallel_loop(0, N // 16, unroll=8)
    def _(g):
        for d in range(DSL):
            acc_t[d, pl.ds(g * 16, 16)] = zeros

    bufs = [pk_b0, pk_b1]
    sems = [sem0, sem1]
    copies = [first, None]
    for ci in range(NCHUNK):
        if ci + 1 < NCHUNK:
            copies[(ci + 1) % 2] = pltpu.async_copy(
                pk_hbm.at[0, pl.ds((ci + 1) * CHUNK, CHUNK)],
                bufs[(ci + 1) % 2], sems[(ci + 1) % 2])
        copies[ci % 2].wait()
        pk_b = bufs[ci % 2]

        @plsc.parallel_loop(0, CHUNK, step=16, unroll=16)
        def _(b):
            pk_v = pk_b[pl.ds(b, 16)]
            src_v = pk_v & 0xFFFF
            dst_v = pk_v >> 16
            for d in range(DSL):
                vals = plsc.load_gather(xn_t.at[d], [src_v])
                plsc.addupdate_scatter(acc_t.at[d], [dst_v], vals)

    pltpu.sync_copy(acc_t, out_hbm.at[pl.ds(d0, DSL), :])


_sc_scatter = functools.partial(
    pl.kernel,
    out_type=_tr,
    mesh=plsc.VectorSubcoreMesh(core_axis_name="c", subcore_axis_name="s"),
    compiler_params=pltpu.CompilerParams(use_tc_tiling_on_sc=False,
                                         needs_layout_passes=False),
    scratch_types=[
        pltpu.VMEM((DSL, N), jnp.float32),
        pltpu.VMEM((DSL, N), jnp.float32),
        pltpu.VMEM((CHUNK,), jnp.int32),
        pltpu.VMEM((CHUNK,), jnp.int32),
        pltpu.SemaphoreType.DMA,
        pltpu.SemaphoreType.DMA,
    ],
)(_scatter_body)


def kernel(x, edge_index, Wl1, bl1, Wr1, scale1, Wl2, bl2, Wr2, scale2):
    wl1ts = (Wl1 * scale1).T
    wl2ts = (Wl2 * scale2).T
    wr1t = Wr1.T
    wr2t = Wr2.T
    bl1r = bl1.reshape(1, D)
    bl2r = bl2.reshape(1, D)

    pk = _pack_edges(edge_index)
    xnt = _normalize(x)
    pt1 = _sc_scatter(xnt, pk)
    h, hnt = _mid(x, pt1, wl1ts, bl1r, wr1t)
    pt2 = _sc_scatter(hnt, pk)
    return _final(h, pt2, wl2ts, bl2r, wr2t)
